# initial kernel scaffold (unmeasured)
import jax
import jax.numpy as jnp
from jax import lax
from jax.experimental import pallas as pl
from jax.experimental.pallas import tpu as pltpu

N_DEV = 4


def kernel(x, w_mat, scale_x, scale_w):
    M, K = x.shape
    _, N = w_mat.shape
    CM = M // N_DEV

    scale = (scale_x.reshape(1, 1) * scale_w.reshape(1, 1)).astype(jnp.float32)

    def body(scale_ref, x_ref, w_ref, out_ref,
             send_buf, recv_buf, send_sems, recv_sems):
        i = lax.axis_index("i")
        left = (i + N_DEV - 1) % N_DEV
        right = (i + 1) % N_DEV

        barrier_sem = pltpu.get_barrier_semaphore()
        for nbr in [left, right]:
            pl.semaphore_signal(
                barrier_sem, inc=1,
                device_id=(nbr,), device_id_type=pl.DeviceIdType.MESH,
            )
        pl.semaphore_wait(barrier_sem, 2)

        s = scale_ref[0, 0]
        out_ref[:, :] = (
            jnp.dot(x_ref[:, :], w_ref[:, :],
                    preferred_element_type=jnp.float32) * s
        )

        def rows(c):
            return pl.ds(c * CM, CM)

        for h in range(N_DEV - 1):
            c_send = (i - h + N_DEV) % N_DEV
            acc = out_ref[rows(c_send), :]
            if h > 0:
                acc = acc + recv_buf[h - 1].astype(jnp.float32)
            send_buf[:, :] = acc.astype(jnp.bfloat16)
            rdma = pltpu.make_async_remote_copy(
                src_ref=send_buf,
                dst_ref=recv_buf.at[h],
                send_sem=send_sems.at[h],
                recv_sem=recv_sems.at[h],
                device_id=(right,),
                device_id_type=pl.DeviceIdType.MESH,
            )
            rdma.start()
            rdma.wait()

        own = (i + 1) % N_DEV
        reduced = out_ref[rows(own), :] + recv_buf[N_DEV - 2].astype(jnp.float32)
        out_ref[rows(own), :] = reduced

        send_buf[:, :] = reduced.astype(jnp.bfloat16)
        for t in range(N_DEV - 1):
            slot = (N_DEV - 1) + t
            src = send_buf if t == 0 else recv_buf.at[slot - 1]
            rdma = pltpu.make_async_remote_copy(
                src_ref=src,
                dst_ref=recv_buf.at[slot],
                send_sem=send_sems.at[slot],
                recv_sem=recv_sems.at[slot],
                device_id=(right,),
                device_id_type=pl.DeviceIdType.MESH,
            )
            rdma.start()
            rdma.wait()
            c_recv = (i - t + N_DEV) % N_DEV
            out_ref[rows(c_recv), :] = recv_buf[slot].astype(jnp.float32)

    return pl.pallas_call(
        body,
        out_shape=jax.ShapeDtypeStruct((M, N), jnp.float32),
        in_specs=[
            pl.BlockSpec(memory_space=pltpu.SMEM),
            pl.BlockSpec(memory_space=pltpu.VMEM),
            pl.BlockSpec(memory_space=pltpu.VMEM),
        ],
        out_specs=pl.BlockSpec(memory_space=pltpu.VMEM),
        scratch_shapes=[
            pltpu.VMEM((CM, N), jnp.bfloat16),
            pltpu.VMEM((6, CM, N), jnp.bfloat16),
            pltpu.SemaphoreType.DMA((6,)),
            pltpu.SemaphoreType.DMA((6,)),
        ],
        compiler_params=pltpu.CompilerParams(collective_id=0),
    )(scale, x, w_mat)


# baseline (device time: 351582 ns/iter reference)
import jax
import jax.numpy as jnp
from jax import lax
from jax.experimental import pallas as pl
from jax.experimental.pallas import tpu as pltpu

N_DEV = 4
N_HOPS = 2 * (N_DEV - 1)
N_SLOTS = 4


def kernel(x, w_mat, scale_x, scale_w):
    M, K = x.shape
    _, N = w_mat.shape
    CM = M // N_DEV

    scale = (scale_x.reshape(1, 1) * scale_w.reshape(1, 1)).astype(jnp.float32)

    def body(scale_ref, x_hbm, w_hbm, out_hbm,
             w_vmem, x_chunk, stage, send_buf, recv_buf,
             send_sems, recv_sems, load_sem, store_sem):
        i = lax.axis_index("i")
        left = (i + N_DEV - 1) % N_DEV
        right = (i + 1) % N_DEV

        barrier_sem = pltpu.get_barrier_semaphore()
        for nbr in [left, right]:
            pl.semaphore_signal(
                barrier_sem, inc=1,
                device_id=(nbr,), device_id_type=pl.DeviceIdType.MESH,
            )
        pl.semaphore_wait(barrier_sem, 2)

        cp = pltpu.make_async_copy(w_hbm, w_vmem, load_sem)
        cp.start()
        cp.wait()
        s = scale_ref[0, 0]

        def load_x(c):
            cp = pltpu.make_async_copy(
                x_hbm.at[pl.ds(c * CM, CM), :], x_chunk, load_sem)
            cp.start()
            cp.wait()

        def partial():
            return jnp.dot(
                x_chunk[:, :].astype(jnp.bfloat16),
                w_vmem[:, :].astype(jnp.bfloat16),
                preferred_element_type=jnp.float32,
            ) * s

        def hop(h, src):
            rdma = pltpu.make_async_remote_copy(
                src_ref=src,
                dst_ref=recv_buf.at[h % N_SLOTS],
                send_sem=send_sems.at[h],
                recv_sem=recv_sems.at[h],
                device_id=(right,),
                device_id_type=pl.DeviceIdType.MESH,
            )
            rdma.start()
            rdma.wait()

        def store_out(c):
            cp = pltpu.make_async_copy(
                stage, out_hbm.at[pl.ds(c * CM, CM), :], store_sem)
            cp.start()
            cp.wait()

        for h in range(N_DEV - 1):
            load_x((i - h + N_DEV) % N_DEV)
            p = partial()
            if h > 0:
                p = p + recv_buf[(h - 1) % N_SLOTS].astype(jnp.float32)
            send_buf[:, :] = p.astype(jnp.bfloat16)
            hop(h, send_buf)

        own = (i + 1) % N_DEV
        load_x(own)
        stage[:, :] = (
            partial() + recv_buf[(N_DEV - 2) % N_SLOTS].astype(jnp.float32)
        )
        send_buf[:, :] = stage[:, :].astype(jnp.bfloat16)
        store_out(own)

        for t in range(N_DEV - 1):
            h = (N_DEV - 1) + t
            src = send_buf if t == 0 else recv_buf.at[(h - 1) % N_SLOTS]
            hop(h, src)
            stage[:, :] = recv_buf[h % N_SLOTS].astype(jnp.float32)
            store_out((i - t + N_DEV) % N_DEV)

    return pl.pallas_call(
        body,
        out_shape=jax.ShapeDtypeStruct((M, N), jnp.float32),
        in_specs=[
            pl.BlockSpec(memory_space=pltpu.SMEM),
            pl.BlockSpec(memory_space=pl.ANY),
            pl.BlockSpec(memory_space=pl.ANY),
        ],
        out_specs=pl.BlockSpec(memory_space=pl.ANY),
        scratch_shapes=[
            pltpu.VMEM((K, N), jnp.float32),
            pltpu.VMEM((CM, K), jnp.float32),
            pltpu.VMEM((CM, N), jnp.float32),
            pltpu.VMEM((CM, N), jnp.bfloat16),
            pltpu.VMEM((N_SLOTS, CM, N), jnp.bfloat16),
            pltpu.SemaphoreType.DMA((N_HOPS,)),
            pltpu.SemaphoreType.DMA((N_HOPS,)),
            pltpu.SemaphoreType.DMA,
            pltpu.SemaphoreType.DMA,
        ],
        compiler_params=pltpu.CompilerParams(
            collective_id=0, vmem_limit_bytes=60 * 1024 * 1024),
    )(scale, x, w_mat)


# device time: 186462 ns/iter; 1.8855x vs baseline; 1.8855x over previous
import jax
import jax.numpy as jnp
from jax import lax
from jax.experimental import pallas as pl
from jax.experimental.pallas import tpu as pltpu

N_DEV = 4
N_HOPS = 2 * (N_DEV - 1)
N_SLOTS = 4


def kernel(x, w_mat, scale_x, scale_w):
    M, K = x.shape
    _, N = w_mat.shape
    CM = M // N_DEV
    NH = N // 2

    bf16 = jnp.bfloat16
    f32 = jnp.float32

    scale = (scale_x.reshape(1, 1) * scale_w.reshape(1, 1)).astype(f32)

    def body(scale_ref, x_hbm, w_hbm, out_hbm,
             w_vmem, x_bufs, acc, send_a, send_b, recv_a, recv_b,
             ssem_a, rsem_a, ssem_b, rsem_b,
             load_sems, store_sems):
        i = lax.axis_index("i")
        left = (i + N_DEV - 1) % N_DEV
        right = (i + 1) % N_DEV

        barrier_sem = pltpu.get_barrier_semaphore()
        for nbr in [left, right]:
            pl.semaphore_signal(
                barrier_sem, inc=1,
                device_id=(nbr,), device_id_type=pl.DeviceIdType.MESH,
            )
        pl.semaphore_wait(barrier_sem, 2)

        wcp = pltpu.make_async_copy(w_hbm, w_vmem, load_sems.at[2])
        wcp.start()
        s = scale_ref[0, 0]

        def load_x(c, slot):
            cp = pltpu.make_async_copy(
                x_hbm.at[pl.ds(c * CM, CM), :], x_bufs.at[slot],
                load_sems.at[slot])
            cp.start()
            return cp

        def gemm(slot, cols):
            return jnp.dot(
                x_bufs[slot].astype(bf16),
                w_vmem[:, cols].astype(bf16),
                preferred_element_type=f32,
            ) * s

        A = pl.ds(0, NH)
        B = pl.ds(NH, NH)

        def hop(h, src_a, src_b):
            a = pltpu.make_async_remote_copy(
                src_ref=src_a, dst_ref=recv_a.at[h % N_SLOTS],
                send_sem=ssem_a.at[h], recv_sem=rsem_a.at[h],
                device_id=(right,), device_id_type=pl.DeviceIdType.MESH)
            b = pltpu.make_async_remote_copy(
                src_ref=src_b, dst_ref=recv_b.at[h % N_SLOTS],
                send_sem=ssem_b.at[h], recv_sem=rsem_b.at[h],
                device_id=(left,), device_id_type=pl.DeviceIdType.MESH)
            a.start()
            b.start()
            return a, b

        def store(c, cols, sem_idx):
            cp = pltpu.make_async_copy(
                acc.at[:, cols],
                out_hbm.at[pl.ds(c * CM, CM), cols],
                store_sems.at[sem_idx])
            cp.start()
            return cp

        xcp = load_x(i, 0)
        wcp.wait()
        xcp.wait()
        acc[:, :] = gemm(0, pl.ds(0, N))
        send_a[:, :] = acc[:, A].astype(bf16)
        send_b[:, :] = acc[:, B].astype(bf16)

        for h in range(N_DEV - 1):
            a, b = hop(h, send_a, send_b)
            if h == 0:
                ca = load_x((i - 1 + N_DEV) % N_DEV, 0)
                cb = load_x((i + 1) % N_DEV, 1)
                ca.wait()
                acc[:, A] = gemm(0, A)
                cb.wait()
                acc[:, B] = gemm(1, B)
            elif h == 1:
                c = load_x((i + 2) % N_DEV, 0)
                c.wait()
                acc[:, :] = gemm(0, pl.ds(0, N))
            else:
                ca = load_x((i + 1) % N_DEV, 0)
                cb = load_x((i - 1 + N_DEV) % N_DEV, 1)
                ca.wait()
                acc[:, A] = gemm(0, A)
                cb.wait()
                acc[:, B] = gemm(1, B)
            a.wait_recv()
            b.wait_recv()
            a.wait_send()
            b.wait_send()
            if h < N_DEV - 2:
                send_a[:, :] = (acc[:, A]
                                + recv_a[h % N_SLOTS].astype(f32)).astype(bf16)
                send_b[:, :] = (acc[:, B]
                                + recv_b[h % N_SLOTS].astype(f32)).astype(bf16)
            else:
                acc[:, A] = acc[:, A] + recv_a[h % N_SLOTS].astype(f32)
                acc[:, B] = acc[:, B] + recv_b[h % N_SLOTS].astype(f32)
                send_a[:, :] = acc[:, A].astype(bf16)
                send_b[:, :] = acc[:, B].astype(bf16)

        a, b = hop(N_DEV - 1, send_a, send_b)
        st_a = store((i + 1) % N_DEV, A, 0)
        st_b = store((i - 1 + N_DEV) % N_DEV, B, 1)
        ag = [(a, b)]
        for t in range(N_DEV - 1):
            h = (N_DEV - 1) + t
            a, b = ag[-1]
            a.wait_recv()
            b.wait_recv()
            if t < N_DEV - 2:
                ag.append(hop(h + 1, recv_a.at[h % N_SLOTS],
                              recv_b.at[h % N_SLOTS]))
            st_a.wait()
            acc[:, A] = recv_a[h % N_SLOTS].astype(f32)
            st_a = store((i - t + N_DEV) % N_DEV, A, 0)
            st_b.wait()
            acc[:, B] = recv_b[h % N_SLOTS].astype(f32)
            st_b = store((i + t) % N_DEV, B, 1)
        for a, b in ag:
            a.wait_send()
            b.wait_send()
        st_a.wait()
        st_b.wait()

    return pl.pallas_call(
        body,
        out_shape=jax.ShapeDtypeStruct((M, N), f32),
        in_specs=[
            pl.BlockSpec(memory_space=pltpu.SMEM),
            pl.BlockSpec(memory_space=pl.ANY),
            pl.BlockSpec(memory_space=pl.ANY),
        ],
        out_specs=pl.BlockSpec(memory_space=pl.ANY),
        scratch_shapes=[
            pltpu.VMEM((K, N), f32),
            pltpu.VMEM((2, CM, K), f32),
            pltpu.VMEM((CM, N), f32),
            pltpu.VMEM((CM, NH), bf16),
            pltpu.VMEM((CM, NH), bf16),
            pltpu.VMEM((N_SLOTS, CM, NH), bf16),
            pltpu.VMEM((N_SLOTS, CM, NH), bf16),
            pltpu.SemaphoreType.DMA((N_HOPS,)),
            pltpu.SemaphoreType.DMA((N_HOPS,)),
            pltpu.SemaphoreType.DMA((N_HOPS,)),
            pltpu.SemaphoreType.DMA((N_HOPS,)),
            pltpu.SemaphoreType.DMA((3,)),
            pltpu.SemaphoreType.DMA((2,)),
        ],
        compiler_params=pltpu.CompilerParams(
            collective_id=0, vmem_limit_bytes=60 * 1024 * 1024),
    )(scale, x, w_mat)


# device time: 174805 ns/iter; 2.0113x vs baseline; 1.0667x over previous
import jax
import jax.numpy as jnp
from jax import lax
from jax.experimental import pallas as pl
from jax.experimental.pallas import tpu as pltpu

N_DEV = 4
N_HOPS = 2 * (N_DEV - 1)
N_SLOTS = 4
N_SUB = 2


def kernel(x, w_mat, scale_x, scale_w):
    M, K = x.shape
    _, N = w_mat.shape
    CM = M // N_DEV
    NH = N // 2
    SW = NH // N_SUB

    bf16 = jnp.bfloat16
    f32 = jnp.float32

    scale = (scale_x.reshape(1, 1) * scale_w.reshape(1, 1)).astype(f32)

    def body(scale_ref, x_hbm, w_hbm, out_hbm,
             w_vmem, w_bf, x_bufs, acc, send_bufs, recv_bufs,
             ssems, rsems, load_sems, store_sems):
        i = lax.axis_index("i")
        left = (i + N_DEV - 1) % N_DEV
        right = (i + 1) % N_DEV
        peer = [right, left]

        wcp = pltpu.make_async_copy(w_hbm, w_vmem, load_sems.at[2])
        wcp.start()

        def load_x(c, slot):
            cp = pltpu.make_async_copy(
                x_hbm.at[pl.ds(c * CM, CM), :], x_bufs.at[slot],
                load_sems.at[slot])
            cp.start()
            return cp

        xcp = load_x(i, 0)

        barrier_sem = pltpu.get_barrier_semaphore()
        for nbr in [left, right]:
            pl.semaphore_signal(
                barrier_sem, inc=1,
                device_id=(nbr,), device_id_type=pl.DeviceIdType.MESH,
            )
        pl.semaphore_wait(barrier_sem, 2)

        s = scale_ref[0, 0]

        def gemm(slot, cols):
            return jnp.dot(
                x_bufs[slot].astype(bf16), w_bf[:, cols],
                preferred_element_type=f32,
            ) * s

        def acol(r, u):
            return pl.ds(r * NH + u * SW, SW)

        A = pl.ds(0, NH)
        B = pl.ds(NH, NH)

        def start_send(h, r, u, src):
            rd = pltpu.make_async_remote_copy(
                src_ref=src,
                dst_ref=recv_bufs.at[r, h % N_SLOTS, :, pl.ds(u * SW, SW)],
                send_sem=ssems.at[r, h, u], recv_sem=rsems.at[r, h, u],
                device_id=(peer[r],), device_id_type=pl.DeviceIdType.MESH)
            rd.start()
            return rd

        def store(c, cols, r):
            cp = pltpu.make_async_copy(
                acc.at[:, cols],
                out_hbm.at[pl.ds(c * CM, CM), cols],
                store_sems.at[r])
            cp.start()
            return cp

        wcp.wait()
        w_bf[:, :] = w_vmem[:, :].astype(bf16)
        xcp.wait()
        acc[:, :] = gemm(0, pl.ds(0, N))
        sends = {}
        for u in range(N_SUB):
            for r in range(2):
                sb = pl.ds(u * SW, SW)
                send_bufs[r, :, sb] = acc[:, acol(r, u)].astype(bf16)
                sends[(r, u)] = start_send(
                    0, r, u, send_bufs.at[r, :, sb])

        def process(h, own):
            for u in range(N_SUB):
                for r in range(2):
                    sends[(r, u)].wait_recv()
                    sends[(r, u)].wait_send()
                    sb = pl.ds(u * SW, SW)
                    rv = recv_bufs[r, h % N_SLOTS, :, u * SW:(u + 1) * SW]
                    if not own:
                        send_bufs[r, :, sb] = (
                            acc[:, acol(r, u)] + rv.astype(f32)
                        ).astype(bf16)
                    else:
                        acc[:, acol(r, u)] = acc[:, acol(r, u)] + rv.astype(f32)
                        send_bufs[r, :, sb] = acc[:, acol(r, u)].astype(bf16)
                    sends[(r, u)] = start_send(
                        h + 1, r, u, send_bufs.at[r, :, sb])

        ca = load_x((i - 1 + N_DEV) % N_DEV, 0)
        cb = load_x((i + 1) % N_DEV, 1)
        ca.wait()
        acc[:, A] = gemm(0, A)
        cb.wait()
        acc[:, B] = gemm(1, B)
        process(0, own=False)

        c = load_x((i + 2) % N_DEV, 0)
        c.wait()
        acc[:, :] = gemm(0, pl.ds(0, N))
        process(1, own=False)

        cb = load_x((i - 1 + N_DEV) % N_DEV, 0)
        acc[:, A] = gemm(1, A)
        cb.wait()
        acc[:, B] = gemm(0, B)
        process(2, own=True)

        st = [store((i + 1) % N_DEV, A, 0),
              store((i - 1 + N_DEV) % N_DEV, B, 1)]

        for t in range(N_DEV - 1):
            h = (N_DEV - 1) + t
            for u in range(N_SUB):
                for r in range(2):
                    sends[(r, u)].wait_recv()
                    sends[(r, u)].wait_send()
                    if t < N_DEV - 2:
                        sends[(r, u)] = start_send(
                            h + 1, r, u,
                            recv_bufs.at[r, h % N_SLOTS, :, pl.ds(u * SW, SW)])
            c_of = [(i - t + N_DEV) % N_DEV, (i + t) % N_DEV]
            for r, cols in ((0, A), (1, B)):
                st[r].wait()
                acc[:, cols] = recv_bufs[r, h % N_SLOTS, :, :].astype(f32)
                st[r] = store(c_of[r], cols, r)
        st[0].wait()
        st[1].wait()

    return pl.pallas_call(
        body,
        out_shape=jax.ShapeDtypeStruct((M, N), f32),
        in_specs=[
            pl.BlockSpec(memory_space=pltpu.SMEM),
            pl.BlockSpec(memory_space=pl.ANY),
            pl.BlockSpec(memory_space=pl.ANY),
        ],
        out_specs=pl.BlockSpec(memory_space=pl.ANY),
        scratch_shapes=[
            pltpu.VMEM((K, N), f32),
            pltpu.VMEM((K, N), bf16),
            pltpu.VMEM((2, CM, K), f32),
            pltpu.VMEM((CM, N), f32),
            pltpu.VMEM((2, CM, NH), bf16),
            pltpu.VMEM((2, N_SLOTS, CM, NH), bf16),
            pltpu.SemaphoreType.DMA((2, N_HOPS, N_SUB)),
            pltpu.SemaphoreType.DMA((2, N_HOPS, N_SUB)),
            pltpu.SemaphoreType.DMA((3,)),
            pltpu.SemaphoreType.DMA((2,)),
        ],
        compiler_params=pltpu.CompilerParams(
            collective_id=0, vmem_limit_bytes=62 * 1024 * 1024),
    )(scale, x, w_mat)
